# Initial kernel scaffold; baseline (speedup 1.0000x reference)
import jax
import jax.numpy as jnp
from jax.experimental import pallas as pl


def kernel(query_tokens, support_feats, geo_embed, q_idx, s_idx, num_queries, Wq, Wk, Wv, Wg, Wo, bo, log_tau):
    raise NotImplementedError("write your pallas kernel here")


# speedup vs baseline: 1.7233x; 1.7233x over previous
import functools

import jax
import jax.numpy as jnp
from jax import lax
from jax.experimental import pallas as pl
from jax.experimental.pallas import tpu as pltpu
from jax.experimental.pallas import tpu_sc as plsc

NQ = 10000
NS = 10000
TD = 128
GD = 64
H = 8
HD = 16
E = 320000

NSUB = 16
NW = NSUB
B = 128
NBLK = E // B
GPB = B // 16
ACC_R = 10240
NCHUNK = 4
QH = ACC_R // NCHUNK
CH_R = QH + 8
DH = QH // 16
DH_R = DH + 8
DEN_R = NCHUNK * DH
NPT = QH // NSUB
_SLICES = [(j * 128, 128) for j in range(NPT // 128)]
if NPT % 128:
    _SLICES.append((NPT - NPT % 128, NPT % 128))
DTILES = 4
DROWS = DH // DTILES

_MB = 1000
_GRID = NQ // _MB


def _proj_body(q_ref, s_ref, wq_ref, wk_ref, wv_ref, qf_ref, kf_ref, vf_ref):
    qf_ref[...] = jnp.dot(q_ref[...], wq_ref[...],
                          preferred_element_type=jnp.float32)
    kf_ref[...] = jnp.dot(s_ref[...], wk_ref[...],
                          preferred_element_type=jnp.float32)
    vf_ref[...] = jnp.dot(s_ref[...], wv_ref[...],
                          preferred_element_type=jnp.float32)


_proj_call = pl.pallas_call(
    _proj_body,
    grid=(_GRID,),
    in_specs=[
        pl.BlockSpec((_MB, TD), lambda i: (i, 0)),
        pl.BlockSpec((_MB, TD), lambda i: (i, 0)),
        pl.BlockSpec((TD, TD), lambda i: (0, 0)),
        pl.BlockSpec((TD, TD), lambda i: (0, 0)),
        pl.BlockSpec((TD, TD), lambda i: (0, 0)),
    ],
    out_specs=[
        pl.BlockSpec((_MB, TD), lambda i: (i, 0)),
        pl.BlockSpec((_MB, TD), lambda i: (i, 0)),
        pl.BlockSpec((_MB, TD), lambda i: (i, 0)),
    ],
    out_shape=[
        jax.ShapeDtypeStruct((NQ, TD), jnp.float32),
        jax.ShapeDtypeStruct((NS, TD), jnp.float32),
        jax.ShapeDtypeStruct((NS, TD), jnp.float32),
    ],
)


@functools.partial(
    pl.kernel,
    mesh=plsc.VectorSubcoreMesh(core_axis_name="c", subcore_axis_name="s",
                                num_cores=1),
    compiler_params=pltpu.CompilerParams(needs_layout_passes=False),
    out_type=[
        jax.ShapeDtypeStruct((ACC_R, TD), jnp.float32),
        jax.ShapeDtypeStruct((DEN_R, TD), jnp.float32),
    ],
    scratch_types=[
        pltpu.VMEM((B,), jnp.int32),
        pltpu.VMEM((B,), jnp.int32),
        pltpu.VMEM((B,), jnp.int32),
        pltpu.VMEM((B,), jnp.int32),
        pltpu.VMEM((B, TD), jnp.float32),
        pltpu.VMEM((B, TD), jnp.float32),
        pltpu.VMEM((B, TD), jnp.float32),
        pltpu.VMEM((B, TD), jnp.float32),
        pltpu.VMEM((B, TD), jnp.float32),
        pltpu.VMEM((B, TD), jnp.float32),
        pltpu.VMEM_SHARED((CH_R, TD), jnp.float32),
        pltpu.VMEM_SHARED((DH_R, TD), jnp.float32),
        pltpu.SemaphoreType.DMA,
        pltpu.SemaphoreType.DMA,
        pltpu.SemaphoreType.DMA,
    ],
)
def _edge_kernel(qf, kf, vf, qidx, sidx, onum, oden,
                 qiv, siv, qriv, qdiv, qr, kr, vr, wv, wden, zbuf, accn, accd,
                 semq, semk, semv):
    wid = lax.axis_index("s")
    zero16 = jnp.zeros((16,), jnp.float32)

    def _zrow(ref):
        def body(r, carry):
            for c16 in range(TD // 16):
                ref[r, pl.ds(c16 * 16, 16)] = zero16
            return carry
        lax.fori_loop(0, B, body, 0)

    _zrow(zbuf)
    _zrow(wden)

    nb = jnp.where(wid < NBLK % NW, NBLK // NW + 1, NBLK // NW)

    for chunk in range(NCHUNK):
        q_lo = chunk * QH

        r0 = wid * NPT
        for off, rows_j in _SLICES:
            pltpu.sync_copy(zbuf.at[pl.ds(0, rows_j)],
                            accn.at[pl.ds(r0 + off, rows_j)])

        @pl.when(wid == 0)
        def _():
            pltpu.sync_copy(zbuf.at[pl.ds(0, 8)], accn.at[pl.ds(QH, 8)])

        @pl.when(wid < DTILES)
        def _():
            pltpu.sync_copy(zbuf.at[pl.ds(0, DROWS)],
                            accd.at[pl.ds(wid * DROWS, DROWS)])

        @pl.when(wid == DTILES)
        def _():
            pltpu.sync_copy(zbuf.at[pl.ds(0, 8)], accd.at[pl.ds(DH, 8)])

        plsc.subcore_barrier()

        def _block(b, carry):
            e0 = (wid + NW * b) * B
            pltpu.sync_copy(qidx.at[pl.ds(e0, B)], qiv)
            pltpu.sync_copy(sidx.at[pl.ds(e0, B)], siv)
            cq = pltpu.async_copy(qf.at[qiv], qr, semq)
            ck = pltpu.async_copy(kf.at[siv], kr, semk)
            cv = pltpu.async_copy(vf.at[siv], vr, semv)
            cq.wait()
            ck.wait()
            cv.wait()

            def _group(g, gcarry):
                rows = lax.iota(jnp.int32, 16) + g * 16
                qv16 = qiv[pl.ds(g * 16, 16)]
                rel = qv16 - q_lo
                valid = jnp.logical_and(rel >= 0, rel < QH)
                qriv[pl.ds(g * 16, 16)] = jnp.where(valid, rel, QH)
                qdiv[pl.ds(g * 16, 16)] = jnp.where(
                    valid, lax.shift_right_logical(rel, 4), DH)
                dcol = lax.shift_left(jnp.bitwise_and(qv16, 15), 3)
                for h in range(H):
                    s_acc = jnp.zeros((16,), jnp.float32)
                    for dd in range(HD):
                        col = jnp.full((16,), h * HD + dd, jnp.int32)
                        qv = plsc.load_gather(qr, [rows, col])
                        kv = plsc.load_gather(kr, [rows, col])
                        s_acc = s_acc + qv * kv
                    w = jnp.exp(s_acc)
                    plsc.store_scatter(wden, [rows, dcol + h], w)
                    for dd in range(HD):
                        col = jnp.full((16,), h * HD + dd, jnp.int32)
                        vv = plsc.load_gather(vr, [rows, col])
                        plsc.store_scatter(wv, [rows, col], vv * w)
                return gcarry

            lax.fori_loop(0, GPB, _group, 0)
            pltpu.sync_copy(wv, accn.at[qriv], add=True)
            pltpu.sync_copy(wden, accd.at[qdiv], add=True)

            def _rez(g, gcarry):
                rows = lax.iota(jnp.int32, 16) + g * 16
                qv16 = qiv[pl.ds(g * 16, 16)]
                dcol = lax.shift_left(jnp.bitwise_and(qv16, 15), 3)
                for h in range(H):
                    plsc.store_scatter(wden, [rows, dcol + h], zero16)
                return gcarry

            lax.fori_loop(0, GPB, _rez, 0)
            return carry

        lax.fori_loop(0, nb, _block, 0)
        plsc.subcore_barrier()

        for off, rows_j in _SLICES:
            pltpu.sync_copy(accn.at[pl.ds(r0 + off, rows_j)],
                            onum.at[pl.ds(q_lo + r0 + off, rows_j)])

        @pl.when(wid < DTILES)
        def _():
            pltpu.sync_copy(accd.at[pl.ds(wid * DROWS, DROWS)],
                            oden.at[pl.ds(chunk * DH + wid * DROWS, DROWS)])

        plsc.subcore_barrier()


def _final_body(n0_ref, d0_ref, geo_ref, wg_ref, sel_ref,
                wo_ref, bo_ref, out_ref):
    nv = n0_ref[...]
    den = d0_ref[...]
    denb = jnp.dot(den, sel_ref[...], preferred_element_type=jnp.float32)
    g = jnp.dot(geo_ref[...], wg_ref[...], preferred_element_type=jnp.float32)
    pre = (nv + g * denb) / jnp.maximum(denb, 1e-8)
    out_ref[...] = (jnp.dot(pre, wo_ref[...],
                            preferred_element_type=jnp.float32) + bo_ref[...])


_final_call = pl.pallas_call(
    _final_body,
    grid=(_GRID,),
    in_specs=[
        pl.BlockSpec((_MB, TD), lambda i: (i, 0)),
        pl.BlockSpec((_MB, H), lambda i: (i, 0)),
        pl.BlockSpec((_MB, GD), lambda i: (i, 0)),
        pl.BlockSpec((GD, TD), lambda i: (0, 0)),
        pl.BlockSpec((H, TD), lambda i: (0, 0)),
        pl.BlockSpec((TD, TD), lambda i: (0, 0)),
        pl.BlockSpec((1, TD), lambda i: (0, 0)),
    ],
    out_specs=pl.BlockSpec((_MB, TD), lambda i: (i, 0)),
    out_shape=jax.ShapeDtypeStruct((NQ, TD), jnp.float32),
)


def kernel(query_tokens, support_feats, geo_embed, q_idx, s_idx, num_queries,
           Wq, Wk, Wv, Wg, Wo, bo, log_tau):
    del num_queries
    scale = jnp.exp(-log_tau) * 0.25
    qf, kf, vf = _proj_call(query_tokens, support_feats, Wq * scale, Wk, Wv)
    n0, d0 = _edge_kernel(qf, kf, vf, q_idx, s_idx)
    d0 = d0.reshape(ACC_R, H)
    sel = jnp.repeat(jnp.eye(H, dtype=jnp.float32), HD, axis=1)
    return _final_call(n0, d0, geo_embed, Wg, sel, Wo, bo.reshape(1, TD))


# speedup vs baseline: 2.2977x; 1.3333x over previous
import functools

import jax
import jax.numpy as jnp
from jax import lax
from jax.experimental import pallas as pl
from jax.experimental.pallas import tpu as pltpu
from jax.experimental.pallas import tpu_sc as plsc

NQ = 10000
NS = 10000
TD = 128
GD = 64
H = 8
HD = 16
E = 320000

NSUB = 16
NW = NSUB
B = 128
NBLK = E // B
GPB = B // 16
ACC_R = 10368
NCHUNK = 3
QH = ACC_R // NCHUNK
CH_R = QH + 8
DH = QH // 16
DH_R = DH + 8
DEN_R = NCHUNK * DH
NPT = QH // NSUB
_SLICES = [(j * 128, 128) for j in range(NPT // 128)]
if NPT % 128:
    _SLICES.append((NPT - NPT % 128, NPT % 128))
DTILES = 9
DROWS = DH // DTILES

_MB = 1000
_GRID = NQ // _MB


def _proj_body(q_ref, s_ref, wq_ref, wk_ref, wv_ref, qf_ref, kf_ref, vf_ref):
    qf_ref[...] = jnp.dot(q_ref[...], wq_ref[...],
                          preferred_element_type=jnp.float32)
    kf_ref[...] = jnp.dot(s_ref[...], wk_ref[...],
                          preferred_element_type=jnp.float32)
    vf_ref[...] = jnp.dot(s_ref[...], wv_ref[...],
                          preferred_element_type=jnp.float32)


_proj_call = pl.pallas_call(
    _proj_body,
    grid=(_GRID,),
    in_specs=[
        pl.BlockSpec((_MB, TD), lambda i: (i, 0)),
        pl.BlockSpec((_MB, TD), lambda i: (i, 0)),
        pl.BlockSpec((TD, TD), lambda i: (0, 0)),
        pl.BlockSpec((TD, TD), lambda i: (0, 0)),
        pl.BlockSpec((TD, TD), lambda i: (0, 0)),
    ],
    out_specs=[
        pl.BlockSpec((_MB, TD), lambda i: (i, 0)),
        pl.BlockSpec((_MB, TD), lambda i: (i, 0)),
        pl.BlockSpec((_MB, TD), lambda i: (i, 0)),
    ],
    out_shape=[
        jax.ShapeDtypeStruct((NQ, TD), jnp.float32),
        jax.ShapeDtypeStruct((NS, TD), jnp.float32),
        jax.ShapeDtypeStruct((NS, TD), jnp.float32),
    ],
)


@functools.partial(
    pl.kernel,
    mesh=plsc.VectorSubcoreMesh(core_axis_name="c", subcore_axis_name="s",
                                num_cores=1),
    compiler_params=pltpu.CompilerParams(needs_layout_passes=False),
    out_type=[
        jax.ShapeDtypeStruct((ACC_R, TD), jnp.float32),
        jax.ShapeDtypeStruct((DEN_R, TD), jnp.float32),
    ],
    scratch_types=[
        pltpu.VMEM((B,), jnp.int32),
        pltpu.VMEM((B,), jnp.int32),
        pltpu.VMEM((B,), jnp.int32),
        pltpu.VMEM((B,), jnp.int32),
        pltpu.VMEM((B, TD), jnp.float32),
        pltpu.VMEM((B, TD), jnp.float32),
        pltpu.VMEM((B, TD), jnp.float32),
        pltpu.VMEM((B, TD), jnp.float32),
        pltpu.VMEM((B, TD), jnp.float32),
        pltpu.VMEM((B, TD), jnp.float32),
        pltpu.VMEM_SHARED((CH_R, TD), jnp.float32),
        pltpu.VMEM_SHARED((DH_R, TD), jnp.float32),
        pltpu.SemaphoreType.DMA,
        pltpu.SemaphoreType.DMA,
        pltpu.SemaphoreType.DMA,
    ],
)
def _edge_kernel(qf, kf, vf, qidx, sidx, onum, oden,
                 qiv, siv, qriv, qdiv, qr, kr, vr, wv, wden, zbuf, accn, accd,
                 semq, semk, semv):
    wid = lax.axis_index("s")
    zero16 = jnp.zeros((16,), jnp.float32)

    def _zrow(ref):
        def body(r, carry):
            for c16 in range(TD // 16):
                ref[r, pl.ds(c16 * 16, 16)] = zero16
            return carry
        lax.fori_loop(0, B, body, 0)

    _zrow(zbuf)
    _zrow(wden)

    nb = jnp.where(wid < NBLK % NW, NBLK // NW + 1, NBLK // NW)

    for chunk in range(NCHUNK):
        q_lo = chunk * QH

        r0 = wid * NPT
        for off, rows_j in _SLICES:
            pltpu.sync_copy(zbuf.at[pl.ds(0, rows_j)],
                            accn.at[pl.ds(r0 + off, rows_j)])

        @pl.when(wid == 0)
        def _():
            pltpu.sync_copy(zbuf.at[pl.ds(0, 8)], accn.at[pl.ds(QH, 8)])

        @pl.when(wid < DTILES)
        def _():
            pltpu.sync_copy(zbuf.at[pl.ds(0, DROWS)],
                            accd.at[pl.ds(wid * DROWS, DROWS)])

        @pl.when(wid == DTILES)
        def _():
            pltpu.sync_copy(zbuf.at[pl.ds(0, 8)], accd.at[pl.ds(DH, 8)])

        plsc.subcore_barrier()

        def _block(b, carry):
            e0 = (wid + NW * b) * B
            pltpu.sync_copy(qidx.at[pl.ds(e0, B)], qiv)
            pltpu.sync_copy(sidx.at[pl.ds(e0, B)], siv)
            cq = pltpu.async_copy(qf.at[qiv], qr, semq)
            ck = pltpu.async_copy(kf.at[siv], kr, semk)
            cv = pltpu.async_copy(vf.at[siv], vr, semv)
            cq.wait()
            ck.wait()
            cv.wait()

            def _group(g, gcarry):
                rows = lax.iota(jnp.int32, 16) + g * 16
                qv16 = qiv[pl.ds(g * 16, 16)]
                rel = qv16 - q_lo
                valid = jnp.logical_and(rel >= 0, rel < QH)
                qriv[pl.ds(g * 16, 16)] = jnp.where(valid, rel, QH)
                qdiv[pl.ds(g * 16, 16)] = jnp.where(
                    valid, lax.shift_right_logical(rel, 4), DH)
                dcol = lax.shift_left(jnp.bitwise_and(qv16, 15), 3)
                for h in range(H):
                    s_acc = jnp.zeros((16,), jnp.float32)
                    for dd in range(HD):
                        col = jnp.full((16,), h * HD + dd, jnp.int32)
                        qv = plsc.load_gather(qr, [rows, col])
                        kv = plsc.load_gather(kr, [rows, col])
                        s_acc = s_acc + qv * kv
                    w = jnp.exp(s_acc)
                    plsc.store_scatter(wden, [rows, dcol + h], w)
                    for dd in range(HD):
                        col = jnp.full((16,), h * HD + dd, jnp.int32)
                        vv = plsc.load_gather(vr, [rows, col])
                        plsc.store_scatter(wv, [rows, col], vv * w)
                return gcarry

            lax.fori_loop(0, GPB, _group, 0)
            pltpu.sync_copy(wv, accn.at[qriv], add=True)
            pltpu.sync_copy(wden, accd.at[qdiv], add=True)

            def _rez(g, gcarry):
                rows = lax.iota(jnp.int32, 16) + g * 16
                qv16 = qiv[pl.ds(g * 16, 16)]
                dcol = lax.shift_left(jnp.bitwise_and(qv16, 15), 3)
                for h in range(H):
                    plsc.store_scatter(wden, [rows, dcol + h], zero16)
                return gcarry

            lax.fori_loop(0, GPB, _rez, 0)
            return carry

        lax.fori_loop(0, nb, _block, 0)
        plsc.subcore_barrier()

        for off, rows_j in _SLICES:
            pltpu.sync_copy(accn.at[pl.ds(r0 + off, rows_j)],
                            onum.at[pl.ds(q_lo + r0 + off, rows_j)])

        @pl.when(wid < DTILES)
        def _():
            pltpu.sync_copy(accd.at[pl.ds(wid * DROWS, DROWS)],
                            oden.at[pl.ds(chunk * DH + wid * DROWS, DROWS)])

        plsc.subcore_barrier()


def _final_body(n0_ref, d0_ref, geo_ref, wg_ref, sel_ref,
                wo_ref, bo_ref, out_ref):
    nv = n0_ref[...]
    den = d0_ref[...]
    denb = jnp.dot(den, sel_ref[...], preferred_element_type=jnp.float32)
    g = jnp.dot(geo_ref[...], wg_ref[...], preferred_element_type=jnp.float32)
    pre = (nv + g * denb) / jnp.maximum(denb, 1e-8)
    out_ref[...] = (jnp.dot(pre, wo_ref[...],
                            preferred_element_type=jnp.float32) + bo_ref[...])


_final_call = pl.pallas_call(
    _final_body,
    grid=(_GRID,),
    in_specs=[
        pl.BlockSpec((_MB, TD), lambda i: (i, 0)),
        pl.BlockSpec((_MB, H), lambda i: (i, 0)),
        pl.BlockSpec((_MB, GD), lambda i: (i, 0)),
        pl.BlockSpec((GD, TD), lambda i: (0, 0)),
        pl.BlockSpec((H, TD), lambda i: (0, 0)),
        pl.BlockSpec((TD, TD), lambda i: (0, 0)),
        pl.BlockSpec((1, TD), lambda i: (0, 0)),
    ],
    out_specs=pl.BlockSpec((_MB, TD), lambda i: (i, 0)),
    out_shape=jax.ShapeDtypeStruct((NQ, TD), jnp.float32),
)


def kernel(query_tokens, support_feats, geo_embed, q_idx, s_idx, num_queries,
           Wq, Wk, Wv, Wg, Wo, bo, log_tau):
    del num_queries
    scale = jnp.exp(-log_tau) * 0.25
    qf, kf, vf = _proj_call(query_tokens, support_feats, Wq * scale, Wk, Wv)
    n0, d0 = _edge_kernel(qf, kf, vf, q_idx, s_idx)
    d0 = d0.reshape(ACC_R, H)
    sel = jnp.repeat(jnp.eye(H, dtype=jnp.float32), HD, axis=1)
    return _final_call(n0, d0, geo_embed, Wg, sel, Wo, bo.reshape(1, TD))


# speedup vs baseline: 2.3393x; 1.0181x over previous
import functools

import jax
import jax.numpy as jnp
from jax import lax
from jax.experimental import pallas as pl
from jax.experimental.pallas import tpu as pltpu
from jax.experimental.pallas import tpu_sc as plsc

NQ = 10000
NS = 10000
TD = 128
GD = 64
H = 8
HD = 16
E = 320000

NSUB = 16
NW = NSUB
B = 128
NBLK = E // B
GPB = B // 16
ACC_R = 10368
NCHUNK = 3
QH = ACC_R // NCHUNK
CH_R = QH + 8
DH = QH // 16
DH_R = DH + 8
DEN_R = NCHUNK * DH
NPT = QH // NSUB
_SLICES = [(j * 128, 128) for j in range(NPT // 128)]
if NPT % 128:
    _SLICES.append((NPT - NPT % 128, NPT % 128))
DTILES = 9
DROWS = DH // DTILES

_MB = 1000
_GRID = NQ // _MB


def _proj_body(q_ref, s_ref, wq_ref, wk_ref, wv_ref, qf_ref, kf_ref, vf_ref):
    qf_ref[...] = jnp.dot(q_ref[...], wq_ref[...],
                          preferred_element_type=jnp.float32)
    kf_ref[...] = jnp.dot(s_ref[...], wk_ref[...],
                          preferred_element_type=jnp.float32)
    vf_ref[...] = jnp.dot(s_ref[...], wv_ref[...],
                          preferred_element_type=jnp.float32)


_proj_call = pl.pallas_call(
    _proj_body,
    grid=(_GRID,),
    in_specs=[
        pl.BlockSpec((_MB, TD), lambda i: (i, 0)),
        pl.BlockSpec((_MB, TD), lambda i: (i, 0)),
        pl.BlockSpec((TD, TD), lambda i: (0, 0)),
        pl.BlockSpec((TD, TD), lambda i: (0, 0)),
        pl.BlockSpec((TD, TD), lambda i: (0, 0)),
    ],
    out_specs=[
        pl.BlockSpec((_MB, TD), lambda i: (i, 0)),
        pl.BlockSpec((_MB, TD), lambda i: (i, 0)),
        pl.BlockSpec((_MB, TD), lambda i: (i, 0)),
    ],
    out_shape=[
        jax.ShapeDtypeStruct((NQ, TD), jnp.float32),
        jax.ShapeDtypeStruct((NS, TD), jnp.float32),
        jax.ShapeDtypeStruct((NS, TD), jnp.float32),
    ],
)


@functools.partial(
    pl.kernel,
    mesh=plsc.VectorSubcoreMesh(core_axis_name="c", subcore_axis_name="s",
                                num_cores=1),
    compiler_params=pltpu.CompilerParams(needs_layout_passes=False),
    out_type=[
        jax.ShapeDtypeStruct((ACC_R, TD), jnp.float32),
        jax.ShapeDtypeStruct((DEN_R, TD), jnp.float32),
    ],
    scratch_types=[
        pltpu.VMEM((B,), jnp.int32),
        pltpu.VMEM((B,), jnp.int32),
        pltpu.VMEM((B,), jnp.int32),
        pltpu.VMEM((B,), jnp.int32),
        pltpu.VMEM((B, TD), jnp.float32),
        pltpu.VMEM((B, 2 * TD), jnp.float32),
        pltpu.VMEM((B, TD), jnp.float32),
        pltpu.VMEM((B, TD), jnp.float32),
        pltpu.VMEM((B, TD), jnp.float32),
        pltpu.VMEM_SHARED((CH_R, TD), jnp.float32),
        pltpu.VMEM_SHARED((DH_R, TD), jnp.float32),
        pltpu.SemaphoreType.DMA,
        pltpu.SemaphoreType.DMA,
        pltpu.SemaphoreType.DMA,
    ],
)
def _edge_kernel(qf, kv, qidx, sidx, onum, oden,
                 qiv, siv, qriv, qdiv, qr, kvr, wv, wden, zbuf, accn, accd,
                 semq, semk, semv):
    wid = lax.axis_index("s")
    zero16 = jnp.zeros((16,), jnp.float32)

    def _zrow(ref):
        def body(r, carry):
            for c16 in range(TD // 16):
                ref[r, pl.ds(c16 * 16, 16)] = zero16
            return carry
        lax.fori_loop(0, B, body, 0)

    _zrow(zbuf)
    _zrow(wden)

    nb = jnp.where(wid < NBLK % NW, NBLK // NW + 1, NBLK // NW)

    for chunk in range(NCHUNK):
        q_lo = chunk * QH

        r0 = wid * NPT
        for off, rows_j in _SLICES:
            pltpu.sync_copy(zbuf.at[pl.ds(0, rows_j)],
                            accn.at[pl.ds(r0 + off, rows_j)])

        @pl.when(wid == 0)
        def _():
            pltpu.sync_copy(zbuf.at[pl.ds(0, 8)], accn.at[pl.ds(QH, 8)])

        @pl.when(wid < DTILES)
        def _():
            pltpu.sync_copy(zbuf.at[pl.ds(0, DROWS)],
                            accd.at[pl.ds(wid * DROWS, DROWS)])

        @pl.when(wid == DTILES)
        def _():
            pltpu.sync_copy(zbuf.at[pl.ds(0, 8)], accd.at[pl.ds(DH, 8)])

        plsc.subcore_barrier()

        def _block(b, carry):
            e0 = (wid + NW * b) * B
            ci = pltpu.async_copy(qidx.at[pl.ds(e0, B)], qiv, semq)
            cj = pltpu.async_copy(sidx.at[pl.ds(e0, B)], siv, semk)
            ci.wait()
            cj.wait()
            cq = pltpu.async_copy(qf.at[qiv], qr, semq)
            ck = pltpu.async_copy(kv.at[siv], kvr, semk)
            cq.wait()
            ck.wait()

            def _group(g, gcarry):
                rows = lax.iota(jnp.int32, 16) + g * 16
                qv16 = qiv[pl.ds(g * 16, 16)]
                rel = qv16 - q_lo
                valid = jnp.logical_and(rel >= 0, rel < QH)
                qriv[pl.ds(g * 16, 16)] = jnp.where(valid, rel, QH)
                qdiv[pl.ds(g * 16, 16)] = jnp.where(
                    valid, lax.shift_right_logical(rel, 4), DH)
                dcol = lax.shift_left(jnp.bitwise_and(qv16, 15), 3)
                for h in range(H):
                    s_acc = jnp.zeros((16,), jnp.float32)
                    for dd in range(HD):
                        col = jnp.full((16,), h * HD + dd, jnp.int32)
                        qv = plsc.load_gather(qr, [rows, col])
                        kvv = plsc.load_gather(kvr, [rows, col])
                        s_acc = s_acc + qv * kvv
                    w = jnp.exp(s_acc)
                    plsc.store_scatter(wden, [rows, dcol + h], w)
                    for dd in range(HD):
                        colv = jnp.full((16,), TD + h * HD + dd, jnp.int32)
                        col = jnp.full((16,), h * HD + dd, jnp.int32)
                        vv = plsc.load_gather(kvr, [rows, colv])
                        plsc.store_scatter(wv, [rows, col], vv * w)
                return gcarry

            lax.fori_loop(0, GPB, _group, 0)
            s1 = pltpu.async_copy(wv, accn.at[qriv], semv, add=True)
            s2 = pltpu.async_copy(wden, accd.at[qdiv], semv, add=True)
            s1.wait()
            s2.wait()

            def _rez(g, gcarry):
                rows = lax.iota(jnp.int32, 16) + g * 16
                qv16 = qiv[pl.ds(g * 16, 16)]
                dcol = lax.shift_left(jnp.bitwise_and(qv16, 15), 3)
                for h in range(H):
                    plsc.store_scatter(wden, [rows, dcol + h], zero16)
                return gcarry

            lax.fori_loop(0, GPB, _rez, 0)
            return carry

        lax.fori_loop(0, nb, _block, 0)
        plsc.subcore_barrier()

        for off, rows_j in _SLICES:
            pltpu.sync_copy(accn.at[pl.ds(r0 + off, rows_j)],
                            onum.at[pl.ds(q_lo + r0 + off, rows_j)])

        @pl.when(wid < DTILES)
        def _():
            pltpu.sync_copy(accd.at[pl.ds(wid * DROWS, DROWS)],
                            oden.at[pl.ds(chunk * DH + wid * DROWS, DROWS)])

        plsc.subcore_barrier()


def _final_body(n0_ref, d0_ref, geo_ref, wg_ref, sel_ref,
                wo_ref, bo_ref, out_ref):
    nv = n0_ref[...]
    den = d0_ref[...]
    denb = jnp.dot(den, sel_ref[...], preferred_element_type=jnp.float32)
    g = jnp.dot(geo_ref[...], wg_ref[...], preferred_element_type=jnp.float32)
    pre = (nv + g * denb) / jnp.maximum(denb, 1e-8)
    out_ref[...] = (jnp.dot(pre, wo_ref[...],
                            preferred_element_type=jnp.float32) + bo_ref[...])


_final_call = pl.pallas_call(
    _final_body,
    grid=(_GRID,),
    in_specs=[
        pl.BlockSpec((_MB, TD), lambda i: (i, 0)),
        pl.BlockSpec((_MB, H), lambda i: (i, 0)),
        pl.BlockSpec((_MB, GD), lambda i: (i, 0)),
        pl.BlockSpec((GD, TD), lambda i: (0, 0)),
        pl.BlockSpec((H, TD), lambda i: (0, 0)),
        pl.BlockSpec((TD, TD), lambda i: (0, 0)),
        pl.BlockSpec((1, TD), lambda i: (0, 0)),
    ],
    out_specs=pl.BlockSpec((_MB, TD), lambda i: (i, 0)),
    out_shape=jax.ShapeDtypeStruct((NQ, TD), jnp.float32),
)


def kernel(query_tokens, support_feats, geo_embed, q_idx, s_idx, num_queries,
           Wq, Wk, Wv, Wg, Wo, bo, log_tau):
    del num_queries
    scale = jnp.exp(-log_tau) * 0.25
    qf, kf, vf = _proj_call(query_tokens, support_feats, Wq * scale, Wk, Wv)
    n0, d0 = _edge_kernel(qf, jnp.concatenate([kf, vf], axis=1), q_idx, s_idx)
    d0 = d0.reshape(ACC_R, H)
    sel = jnp.repeat(jnp.eye(H, dtype=jnp.float32), HD, axis=1)
    return _final_call(n0, d0, geo_embed, Wg, sel, Wo, bo.reshape(1, TD))


# speedup vs baseline: 2.3905x; 1.0219x over previous
import functools

import jax
import jax.numpy as jnp
from jax import lax
from jax.experimental import pallas as pl
from jax.experimental.pallas import tpu as pltpu
from jax.experimental.pallas import tpu_sc as plsc

NQ = 10000
NS = 10000
TD = 128
GD = 64
H = 8
HD = 16
E = 320000

NSUB = 16
NW = NSUB
B = 128
NBLK = E // B
GPB = B // 16
ACC_R = 10368
NCHUNK = 3
QH = ACC_R // NCHUNK
CH_R = QH + 8
DH = QH // 16
DH_R = DH + 8
DEN_R = NCHUNK * DH
NPT = QH // NSUB
_SLICES = [(j * 128, 128) for j in range(NPT // 128)]
if NPT % 128:
    _SLICES.append((NPT - NPT % 128, NPT % 128))
DTILES = 9
DROWS = DH // DTILES

_MB = 1000
_GRID = NQ // _MB


def _proj_body(q_ref, s_ref, wq_ref, wk_ref, wv_ref, qf_ref, kf_ref, vf_ref):
    qf_ref[...] = jnp.dot(q_ref[...], wq_ref[...],
                          preferred_element_type=jnp.float32)
    kf_ref[...] = jnp.dot(s_ref[...], wk_ref[...],
                          preferred_element_type=jnp.float32)
    vf_ref[...] = jnp.dot(s_ref[...], wv_ref[...],
                          preferred_element_type=jnp.float32)


_proj_call = pl.pallas_call(
    _proj_body,
    grid=(_GRID,),
    in_specs=[
        pl.BlockSpec((_MB, TD), lambda i: (i, 0)),
        pl.BlockSpec((_MB, TD), lambda i: (i, 0)),
        pl.BlockSpec((TD, TD), lambda i: (0, 0)),
        pl.BlockSpec((TD, TD), lambda i: (0, 0)),
        pl.BlockSpec((TD, TD), lambda i: (0, 0)),
    ],
    out_specs=[
        pl.BlockSpec((_MB, TD), lambda i: (i, 0)),
        pl.BlockSpec((_MB, TD), lambda i: (i, 0)),
        pl.BlockSpec((_MB, TD), lambda i: (i, 0)),
    ],
    out_shape=[
        jax.ShapeDtypeStruct((NQ, TD), jnp.float32),
        jax.ShapeDtypeStruct((NS, TD), jnp.float32),
        jax.ShapeDtypeStruct((NS, TD), jnp.float32),
    ],
)


@functools.partial(
    pl.kernel,
    mesh=plsc.VectorSubcoreMesh(core_axis_name="c", subcore_axis_name="s",
                                num_cores=1),
    compiler_params=pltpu.CompilerParams(needs_layout_passes=False),
    out_type=[
        jax.ShapeDtypeStruct((ACC_R, TD), jnp.float32),
        jax.ShapeDtypeStruct((DEN_R, TD), jnp.float32),
    ],
    scratch_types=[
        pltpu.VMEM((B,), jnp.int32),
        pltpu.VMEM((B,), jnp.int32),
        pltpu.VMEM((B,), jnp.int32),
        pltpu.VMEM((B,), jnp.int32),
        pltpu.VMEM((B, TD), jnp.float32),
        pltpu.VMEM((B, 2 * TD), jnp.float32),
        pltpu.VMEM((B, TD), jnp.float32),
        pltpu.VMEM((B, TD), jnp.float32),
        pltpu.VMEM((B, TD), jnp.float32),
        pltpu.VMEM_SHARED((CH_R, TD), jnp.float32),
        pltpu.VMEM_SHARED((DH_R, TD), jnp.float32),
        pltpu.SemaphoreType.DMA,
        pltpu.SemaphoreType.DMA,
        pltpu.SemaphoreType.DMA,
    ],
)
def _edge_kernel(qf, kv, qidx, sidx, onum, oden,
                 qiv, siv, qriv, qdiv, qr, kvr, wv, wden, zbuf, accn, accd,
                 semq, semk, semv):
    wid = lax.axis_index("s")
    zero16 = jnp.zeros((16,), jnp.float32)

    def _zrow(ref):
        def body(r, carry):
            for c16 in range(TD // 16):
                ref[r, pl.ds(c16 * 16, 16)] = zero16
            return carry
        lax.fori_loop(0, B, body, 0)

    _zrow(zbuf)
    _zrow(wden)

    nb = jnp.where(wid < NBLK % NW, NBLK // NW + 1, NBLK // NW)

    for chunk in range(NCHUNK):
        q_lo = chunk * QH

        r0 = wid * NPT
        for off, rows_j in _SLICES:
            pltpu.sync_copy(zbuf.at[pl.ds(0, rows_j)],
                            accn.at[pl.ds(r0 + off, rows_j)])

        @pl.when(wid == 0)
        def _():
            pltpu.sync_copy(zbuf.at[pl.ds(0, 8)], accn.at[pl.ds(QH, 8)])

        @pl.when(wid < DTILES)
        def _():
            pltpu.sync_copy(zbuf.at[pl.ds(0, DROWS)],
                            accd.at[pl.ds(wid * DROWS, DROWS)])

        @pl.when(wid == DTILES)
        def _():
            pltpu.sync_copy(zbuf.at[pl.ds(0, 8)], accd.at[pl.ds(DH, 8)])

        plsc.subcore_barrier()

        def _block(b, carry):
            e0 = (wid + NW * b) * B
            ci = pltpu.async_copy(qidx.at[pl.ds(e0, B)], qiv, semq)
            cj = pltpu.async_copy(sidx.at[pl.ds(e0, B)], siv, semk)
            ci.wait()
            cj.wait()
            cq = pltpu.async_copy(qf.at[qiv], qr, semq)
            ck = pltpu.async_copy(kv.at[siv], kvr, semk)
            cq.wait()
            ck.wait()

            def _group(g, gcarry):
                rows = lax.iota(jnp.int32, 16) + g * 16
                qv16 = qiv[pl.ds(g * 16, 16)]
                rel = qv16 - q_lo
                valid = jnp.logical_and(rel >= 0, rel < QH)
                qriv[pl.ds(g * 16, 16)] = jnp.where(valid, rel, QH)
                qdiv[pl.ds(g * 16, 16)] = jnp.where(
                    valid, lax.shift_right_logical(rel, 4), DH)
                dcol = lax.shift_left(jnp.bitwise_and(qv16, 15), 3)
                for h in range(H):
                    s_a = jnp.zeros((16,), jnp.float32)
                    s_b = jnp.zeros((16,), jnp.float32)
                    for dd in range(0, HD, 2):
                        col = jnp.full((16,), h * HD + dd, jnp.int32)
                        col2 = jnp.full((16,), h * HD + dd + 1, jnp.int32)
                        qv = plsc.load_gather(qr, [rows, col])
                        kvv = plsc.load_gather(kvr, [rows, col])
                        qv2 = plsc.load_gather(qr, [rows, col2])
                        kvv2 = plsc.load_gather(kvr, [rows, col2])
                        s_a = s_a + qv * kvv
                        s_b = s_b + qv2 * kvv2
                    w = jnp.exp(s_a + s_b)
                    plsc.store_scatter(wden, [rows, dcol + h], w)
                    for dd in range(HD):
                        colv = jnp.full((16,), TD + h * HD + dd, jnp.int32)
                        col = jnp.full((16,), h * HD + dd, jnp.int32)
                        vv = plsc.load_gather(kvr, [rows, colv])
                        plsc.store_scatter(wv, [rows, col], vv * w)
                return gcarry

            lax.fori_loop(0, GPB, _group, 0)
            s1 = pltpu.async_copy(wv, accn.at[qriv], semv, add=True)
            s2 = pltpu.async_copy(wden, accd.at[qdiv], semv, add=True)
            s1.wait()
            s2.wait()

            def _rez(g, gcarry):
                rows = lax.iota(jnp.int32, 16) + g * 16
                qv16 = qiv[pl.ds(g * 16, 16)]
                dcol = lax.shift_left(jnp.bitwise_and(qv16, 15), 3)
                for h in range(H):
                    plsc.store_scatter(wden, [rows, dcol + h], zero16)
                return gcarry

            lax.fori_loop(0, GPB, _rez, 0)
            return carry

        lax.fori_loop(0, nb, _block, 0)
        plsc.subcore_barrier()

        for off, rows_j in _SLICES:
            pltpu.sync_copy(accn.at[pl.ds(r0 + off, rows_j)],
                            onum.at[pl.ds(q_lo + r0 + off, rows_j)])

        @pl.when(wid < DTILES)
        def _():
            pltpu.sync_copy(accd.at[pl.ds(wid * DROWS, DROWS)],
                            oden.at[pl.ds(chunk * DH + wid * DROWS, DROWS)])

        plsc.subcore_barrier()


def _final_body(n0_ref, d0_ref, geo_ref, wg_ref, sel_ref,
                wo_ref, bo_ref, out_ref):
    nv = n0_ref[...]
    den = d0_ref[...]
    denb = jnp.dot(den, sel_ref[...], preferred_element_type=jnp.float32)
    g = jnp.dot(geo_ref[...], wg_ref[...], preferred_element_type=jnp.float32)
    pre = (nv + g * denb) / jnp.maximum(denb, 1e-8)
    out_ref[...] = (jnp.dot(pre, wo_ref[...],
                            preferred_element_type=jnp.float32) + bo_ref[...])


_final_call = pl.pallas_call(
    _final_body,
    grid=(_GRID,),
    in_specs=[
        pl.BlockSpec((_MB, TD), lambda i: (i, 0)),
        pl.BlockSpec((_MB, H), lambda i: (i, 0)),
        pl.BlockSpec((_MB, GD), lambda i: (i, 0)),
        pl.BlockSpec((GD, TD), lambda i: (0, 0)),
        pl.BlockSpec((H, TD), lambda i: (0, 0)),
        pl.BlockSpec((TD, TD), lambda i: (0, 0)),
        pl.BlockSpec((1, TD), lambda i: (0, 0)),
    ],
    out_specs=pl.BlockSpec((_MB, TD), lambda i: (i, 0)),
    out_shape=jax.ShapeDtypeStruct((NQ, TD), jnp.float32),
)


def kernel(query_tokens, support_feats, geo_embed, q_idx, s_idx, num_queries,
           Wq, Wk, Wv, Wg, Wo, bo, log_tau):
    del num_queries
    scale = jnp.exp(-log_tau) * 0.25
    qf, kf, vf = _proj_call(query_tokens, support_feats, Wq * scale, Wk, Wv)
    n0, d0 = _edge_kernel(qf, jnp.concatenate([kf, vf], axis=1), q_idx, s_idx)
    d0 = d0.reshape(ACC_R, H)
    sel = jnp.repeat(jnp.eye(H, dtype=jnp.float32), HD, axis=1)
    return _final_call(n0, d0, geo_embed, Wg, sel, Wo, bo.reshape(1, TD))
